# baseline (speedup 1.0000x reference)
import functools

import jax
import jax.numpy as jnp
from jax import lax
from jax.experimental import pallas as pl
from jax.experimental.pallas import tpu as pltpu
from jax.experimental.pallas import tpu_sc as plsc

V = 10000
E = 320000
D = 128
VPAD = 10112
NSUB = 16
NCORE = 2
RPS = VPAD // NSUB
SEG = 3200
NSEG = E // SEG
GCH = 64
LCAP = SEG + 144
PADROW = 636
LANE = 16


def _matmuls(verts_pad, Wst, bst):
    def body(v_ref, w_ref, b_ref, o_ref):
        o_ref[0] = (
            jnp.dot(v_ref[...], w_ref[0], preferred_element_type=jnp.float32)
            + b_ref[0]
        )

    BR = 2528
    return pl.pallas_call(
        body,
        grid=(3, VPAD // BR),
        in_specs=[
            pl.BlockSpec((BR, D), lambda j, i: (i, 0)),
            pl.BlockSpec((1, D, D), lambda j, i: (j, 0, 0)),
            pl.BlockSpec((1, 1, D), lambda j, i: (j, 0, 0)),
        ],
        out_specs=pl.BlockSpec((1, BR, D), lambda j, i: (j, i, 0)),
        out_shape=jax.ShapeDtypeStruct((3, VPAD, D), jnp.float32),
    )(verts_pad, Wst, bst)


def _sc_body(init_hbm, table_hbm, ia_hbm, ib_hbm, out_hbm,
             acc, sa, sb, la, lb, rows0, rows1, gsem0, gsem1):
    c = lax.axis_index("c")
    s = lax.axis_index("s")
    lo = s * RPS
    lane = lax.broadcasted_iota(jnp.int32, (LANE,), 0)

    pltpu.sync_copy(init_hbm.at[c, pl.ds(lo, RPS)], acc.at[pl.ds(0, RPS)])

    def accum(rows_ref, base):
        for g in range(4):
            sv16 = la[pl.ds(base + g * LANE, LANE)]
            a16 = sv16 >> 14
            for i in range(LANE):
                r_vec = lax.gather(
                    a16, jnp.full((LANE, 1), i, jnp.int32),
                    dimension_numbers=lax.GatherDimensionNumbers(
                        offset_dims=(), collapsed_slice_dims=(0,),
                        start_index_map=(0,)),
                    slice_sizes=(1,),
                    mode=lax.GatherScatterMode.PROMISE_IN_BOUNDS)
                for k in range(D // LANE):
                    xv = rows_ref[g * LANE + i, pl.ds(k * LANE, LANE)]
                    plsc.addupdate_scatter(acc, [r_vec, lane + k * LANE], xv)

    def seg_body(g, carry):
        pltpu.sync_copy(ia_hbm.at[c, pl.ds(g * SEG, SEG)], sa)
        pltpu.sync_copy(ib_hbm.at[c, pl.ds(g * SEG, SEG)], sb)

        def scan_body(u, off):
            a = sa[pl.ds(u * LANE, LANE)]
            b = sb[pl.ds(u * LANE, LANE)]
            m = (a >= lo) & (a < lo + RPS)
            key = jnp.where(m, jnp.int32(0), jnp.int32(1))
            val = ((a - lo) << 14) | b
            sv = plsc.sort_key_val(key, val)[1]
            la[pl.ds(off, LANE)] = sv
            lb[pl.ds(off, LANE)] = sv & 16383
            cnt = plsc.all_reduce_population_count(m)[0]
            return off + cnt

        n = lax.fori_loop(0, SEG // LANE, scan_body, 0)

        def pad_body(p, x):
            la[pl.ds(n + p * LANE, LANE)] = jnp.full(
                (LANE,), PADROW << 14, jnp.int32)
            lb[pl.ds(n + p * LANE, LANE)] = jnp.zeros((LANE,), jnp.int32)
            return x

        lax.fori_loop(0, (2 * GCH) // LANE, pad_body, 0)
        nb = (n + 2 * GCH - 1) // (2 * GCH)

        @pl.when(nb > 0)
        def _():
            pltpu.async_copy(table_hbm.at[lb.at[pl.ds(0, GCH)]], rows0, gsem0)

        def proc_body(q, x):
            base = q * 2 * GCH
            pltpu.async_copy(
                table_hbm.at[lb.at[pl.ds(base + GCH, GCH)]], rows1, gsem1)
            pltpu.make_async_copy(
                table_hbm.at[pl.ds(0, GCH)], rows0, gsem0).wait()
            accum(rows0, base)

            @pl.when(q + 1 < nb)
            def _():
                pltpu.async_copy(
                    table_hbm.at[lb.at[pl.ds(base + 2 * GCH, GCH)]],
                    rows0, gsem0)

            pltpu.make_async_copy(
                table_hbm.at[pl.ds(0, GCH)], rows1, gsem1).wait()
            accum(rows1, base + GCH)
            return x

        lax.fori_loop(0, nb, proc_body, 0)
        return carry

    lax.fori_loop(0, NSEG, seg_body, 0)
    pltpu.sync_copy(acc.at[pl.ds(0, RPS)], out_hbm.at[c, pl.ds(lo, RPS)])


def _sc_scatter(init, table, ia2, ib2):
    mesh = plsc.VectorSubcoreMesh(core_axis_name="c", subcore_axis_name="s")
    f = pl.kernel(
        _sc_body,
        out_type=jax.ShapeDtypeStruct((NCORE, VPAD, D), jnp.float32),
        mesh=mesh,
        compiler_params=pltpu.CompilerParams(needs_layout_passes=False),
        scratch_types=[
            pltpu.VMEM((640, D), jnp.float32),
            pltpu.VMEM((SEG,), jnp.int32),
            pltpu.VMEM((SEG,), jnp.int32),
            pltpu.VMEM((LCAP,), jnp.int32),
            pltpu.VMEM((LCAP,), jnp.int32),
            pltpu.VMEM((GCH, D), jnp.float32),
            pltpu.VMEM((GCH, D), jnp.float32),
            pltpu.SemaphoreType.DMA,
            pltpu.SemaphoreType.DMA,
        ],
    )
    return f(init, table, ia2, ib2)


def _add(a, b):
    def body(a_ref, b_ref, o_ref):
        o_ref[...] = a_ref[...] + b_ref[...]

    BR = 2000
    return pl.pallas_call(
        body,
        grid=(V // BR,),
        in_specs=[
            pl.BlockSpec((BR, D), lambda i: (i, 0)),
            pl.BlockSpec((BR, D), lambda i: (i, 0)),
        ],
        out_specs=pl.BlockSpec((BR, D), lambda i: (i, 0)),
        out_shape=jax.ShapeDtypeStruct((V, D), jnp.float32),
    )(a, b)


def kernel(verts, edges, W0, b0, W1, b1):
    verts_pad = jnp.zeros((VPAD, D), jnp.float32).at[:V].set(verts)
    Wst = jnp.stack([W0, jnp.zeros_like(W0), W1])
    bst = jnp.stack([b0, jnp.zeros_like(b0), b1]).reshape(3, 1, D)
    O = _matmuls(verts_pad, Wst, bst)
    init = O[:2]
    table = O[2]

    e = edges.astype(jnp.int32)
    ia2 = jnp.stack([e[:, 0], e[:, 1]])
    ib2 = jnp.stack([e[:, 1], e[:, 0]])

    partials = _sc_scatter(init, table, ia2, ib2)
    return _add(partials[0, :V], partials[1, :V])


# speedup vs baseline: 8.5406x; 8.5406x over previous
import functools

import jax
import jax.numpy as jnp
from jax import lax
from jax.experimental import pallas as pl
from jax.experimental.pallas import tpu as pltpu
from jax.experimental.pallas import tpu_sc as plsc

V = 10000
E = 320000
D = 128
VPAD = 10112
NSUB = 16
NCORE = 2
CHUNK = 128
KB = 16
NBLK = 10
PER_SUB = CHUNK * KB * NBLK
EPAD = PER_SUB * NSUB
RPS = VPAD // NSUB


def _matmuls(verts_pad, Wst, bst):
    def body(v_ref, w_ref, b_ref, o_ref):
        o_ref[0] = (
            jnp.dot(v_ref[...], w_ref[0], preferred_element_type=jnp.float32)
            + b_ref[0]
        )

    BR = 2528
    return pl.pallas_call(
        body,
        grid=(3, VPAD // BR),
        in_specs=[
            pl.BlockSpec((BR, D), lambda j, i: (i, 0)),
            pl.BlockSpec((1, D, D), lambda j, i: (j, 0, 0)),
            pl.BlockSpec((1, 1, D), lambda j, i: (j, 0, 0)),
        ],
        out_specs=pl.BlockSpec((1, BR, D), lambda j, i: (j, i, 0)),
        out_shape=jax.ShapeDtypeStruct((3, VPAD, D), jnp.float32),
    )(verts_pad, Wst, bst)


def _sc_body(of_hbm, ia_hbm, ib_hbm, out_hbm,
             acc, ia_v, ib_v, rows0, rows1, sem0, sem1, sem2, sem3):
    c = lax.axis_index("c")
    s = lax.axis_index("s")
    pltpu.sync_copy(of_hbm.at[pl.ds(c * VPAD + s * RPS, RPS)],
                    acc.at[pl.ds(s * RPS, RPS)])
    plsc.subcore_barrier()

    rows = (rows0, rows1)
    gsems = (sem0, sem1)
    ssems = (sem2, sem3)

    def blk(kb, carry):
        pltpu.sync_copy(ia_hbm.at[c, s, pl.ds(kb * KB, KB)], ia_v)
        pltpu.sync_copy(ib_hbm.at[c, s, pl.ds(kb * KB, KB)], ib_v)
        dg = {0: pltpu.async_copy(of_hbm.at[ib_v.at[0]], rows0, sem0)}
        ds = {}
        for j in range(KB):
            if j + 1 < KB:
                if j >= 1:
                    ds[j - 1].wait()
                dg[j + 1] = pltpu.async_copy(
                    of_hbm.at[ib_v.at[j + 1]], rows[(j + 1) % 2],
                    gsems[(j + 1) % 2])
            dg[j].wait()
            ds[j] = pltpu.async_copy(rows[j % 2], acc.at[ia_v.at[j]],
                                     ssems[j % 2], add=True)
        ds[KB - 2].wait()
        ds[KB - 1].wait()
        return carry

    lax.fori_loop(0, NBLK, blk, 0)
    plsc.subcore_barrier()
    pltpu.sync_copy(acc.at[pl.ds(s * RPS, RPS)],
                    out_hbm.at[c, pl.ds(s * RPS, RPS)])


def _sc_scatter(of, ia3, ib3):
    mesh = plsc.VectorSubcoreMesh(core_axis_name="c", subcore_axis_name="s")
    f = pl.kernel(
        _sc_body,
        out_type=jax.ShapeDtypeStruct((NCORE, VPAD, D), jnp.float32),
        mesh=mesh,
        scratch_types=[
            pltpu.VMEM_SHARED((VPAD, D), jnp.float32),
            pltpu.VMEM((KB, CHUNK), jnp.int32),
            pltpu.VMEM((KB, CHUNK), jnp.int32),
            pltpu.VMEM((CHUNK, D), jnp.float32),
            pltpu.VMEM((CHUNK, D), jnp.float32),
            pltpu.SemaphoreType.DMA,
            pltpu.SemaphoreType.DMA,
            pltpu.SemaphoreType.DMA,
            pltpu.SemaphoreType.DMA,
        ],
    )
    return f(of, ia3, ib3)


def _add(a, b):
    def body(a_ref, b_ref, o_ref):
        o_ref[...] = a_ref[...] + b_ref[...]

    BR = 2000
    return pl.pallas_call(
        body,
        grid=(V // BR,),
        in_specs=[
            pl.BlockSpec((BR, D), lambda i: (i, 0)),
            pl.BlockSpec((BR, D), lambda i: (i, 0)),
        ],
        out_specs=pl.BlockSpec((BR, D), lambda i: (i, 0)),
        out_shape=jax.ShapeDtypeStruct((V, D), jnp.float32),
    )(a, b)


def kernel(verts, edges, W0, b0, W1, b1):
    verts_pad = jnp.zeros((VPAD, D), jnp.float32).at[:V].set(verts)
    Wst = jnp.stack([W0, jnp.zeros_like(W0), W1])
    bst = jnp.stack([b0, jnp.zeros_like(b0), b1]).reshape(3, 1, D)
    Of = _matmuls(verts_pad, Wst, bst).reshape(3 * VPAD, D)

    e = edges.astype(jnp.int32)
    pad = jnp.full((EPAD - E,), V, jnp.int32)
    ia3 = jnp.stack([
        jnp.concatenate([e[:, 0], pad]),
        jnp.concatenate([e[:, 1], pad]),
    ]).reshape(NCORE, NSUB, KB * NBLK, CHUNK)
    ib3 = jnp.stack([
        jnp.concatenate([e[:, 1], pad]),
        jnp.concatenate([e[:, 0], pad]),
    ]).reshape(NCORE, NSUB, KB * NBLK, CHUNK) + 2 * VPAD

    partials = _sc_scatter(Of, ia3, ib3)
    return _add(partials[0, :V], partials[1, :V])


# speedup vs baseline: 8.5448x; 1.0005x over previous
import jax
import jax.numpy as jnp
from jax import lax
from jax.experimental import pallas as pl
from jax.experimental.pallas import tpu as pltpu
from jax.experimental.pallas import tpu_sc as plsc

V = 10000
E = 320000
D = 128
VPAD = 10112
NSUB = 16
NCORE = 2
CHUNK = 128
KB = 16
NBLK = 10
PER_SUB = CHUNK * KB * NBLK
EPAD = PER_SUB * NSUB
RPS = VPAD // NSUB


def _matmuls(verts_pad, Wst, bst):
    def body(v_ref, w_ref, b_ref, o_ref):
        o_ref[0] = (
            jnp.dot(v_ref[...], w_ref[0], preferred_element_type=jnp.float32)
            + b_ref[0]
        )

    BR = 2528
    return pl.pallas_call(
        body,
        grid=(3, VPAD // BR),
        in_specs=[
            pl.BlockSpec((BR, D), lambda j, i: (i, 0)),
            pl.BlockSpec((1, D, D), lambda j, i: (j, 0, 0)),
            pl.BlockSpec((1, 1, D), lambda j, i: (j, 0, 0)),
        ],
        out_specs=pl.BlockSpec((1, BR, D), lambda j, i: (j, i, 0)),
        out_shape=jax.ShapeDtypeStruct((3, VPAD, D), jnp.float32),
    )(verts_pad, Wst, bst)


def _sc_body(of_hbm, ia_hbm, ib_hbm, out_hbm,
             acc, ia_v, ib_v, rows0, rows1, sem0, sem1, sem2, sem3):
    c = lax.axis_index("c")
    s = lax.axis_index("s")
    pltpu.sync_copy(of_hbm.at[pl.ds(c * VPAD + s * RPS, RPS)],
                    acc.at[pl.ds(s * RPS, RPS)])
    plsc.subcore_barrier()

    rows = (rows0, rows1)
    gsems = (sem0, sem1)
    ssems = (sem2, sem3)

    def blk(kb, carry):
        pltpu.sync_copy(ia_hbm.at[c, s, pl.ds(kb * KB, KB)], ia_v)
        pltpu.sync_copy(ib_hbm.at[c, s, pl.ds(kb * KB, KB)], ib_v)
        dg = {0: pltpu.async_copy(of_hbm.at[ib_v.at[0]], rows0, sem0)}
        ds = {}
        for j in range(KB):
            if j + 1 < KB:
                if j >= 1:
                    ds[j - 1].wait()
                dg[j + 1] = pltpu.async_copy(
                    of_hbm.at[ib_v.at[j + 1]], rows[(j + 1) % 2],
                    gsems[(j + 1) % 2])
            dg[j].wait()
            ds[j] = pltpu.async_copy(rows[j % 2], acc.at[ia_v.at[j]],
                                     ssems[j % 2], add=True)
        ds[KB - 2].wait()
        ds[KB - 1].wait()
        return carry

    lax.fori_loop(0, NBLK, blk, 0)
    plsc.subcore_barrier()
    pltpu.sync_copy(acc.at[pl.ds(s * RPS, RPS)],
                    out_hbm.at[c, pl.ds(s * RPS, RPS)])


def _sc_scatter(of, ia3, ib3):
    mesh = plsc.VectorSubcoreMesh(core_axis_name="c", subcore_axis_name="s")
    f = pl.kernel(
        _sc_body,
        out_type=jax.ShapeDtypeStruct((NCORE, VPAD, D), jnp.float32),
        mesh=mesh,
        scratch_types=[
            pltpu.VMEM_SHARED((VPAD, D), jnp.float32),
            pltpu.VMEM((KB, CHUNK), jnp.int32),
            pltpu.VMEM((KB, CHUNK), jnp.int32),
            pltpu.VMEM((CHUNK, D), jnp.float32),
            pltpu.VMEM((CHUNK, D), jnp.float32),
            pltpu.SemaphoreType.DMA,
            pltpu.SemaphoreType.DMA,
            pltpu.SemaphoreType.DMA,
            pltpu.SemaphoreType.DMA,
        ],
    )
    return f(of, ia3, ib3)


def _add(a, b):
    def body(a_ref, b_ref, o_ref):
        o_ref[...] = a_ref[...] + b_ref[...]

    BR = 2000
    return pl.pallas_call(
        body,
        grid=(V // BR,),
        in_specs=[
            pl.BlockSpec((BR, D), lambda i: (i, 0)),
            pl.BlockSpec((BR, D), lambda i: (i, 0)),
        ],
        out_specs=pl.BlockSpec((BR, D), lambda i: (i, 0)),
        out_shape=jax.ShapeDtypeStruct((V, D), jnp.float32),
    )(a, b)


def kernel(verts, edges, W0, b0, W1, b1):
    verts_pad = jnp.zeros((VPAD, D), jnp.float32).at[:V].set(verts)
    Wst = jnp.stack([W0, jnp.zeros_like(W0), W1])
    bst = jnp.stack([b0, jnp.zeros_like(b0), b1]).reshape(3, 1, D)
    Of = _matmuls(verts_pad, Wst, bst).reshape(3 * VPAD, D)

    e = edges.astype(jnp.int32)
    pad = jnp.full((EPAD - E,), V, jnp.int32)
    ia3 = jnp.stack([
        jnp.concatenate([e[:, 0], pad]),
        jnp.concatenate([e[:, 1], pad]),
    ]).reshape(NCORE, NSUB, KB * NBLK, CHUNK)
    ib3 = jnp.stack([
        jnp.concatenate([e[:, 1], pad]),
        jnp.concatenate([e[:, 0], pad]),
    ]).reshape(NCORE, NSUB, KB * NBLK, CHUNK) + 2 * VPAD

    partials = _sc_scatter(Of, ia3, ib3)
    return _add(partials[0, :V], partials[1, :V])


# speedup vs baseline: 8.6382x; 1.0109x over previous
import jax
import jax.numpy as jnp
from jax import lax
from jax.experimental import pallas as pl
from jax.experimental.pallas import tpu as pltpu
from jax.experimental.pallas import tpu_sc as plsc

V = 10000
E = 320000
D = 128
VPAD = 10112
NSUB = 16
NCORE = 2
CHUNK = 128
KB = 40
NBLK = 4
PER_SUB = CHUNK * KB * NBLK
EPAD = PER_SUB * NSUB
RPS = VPAD // NSUB


def _matmuls(verts_pad, Wst, bst):
    def body(v_ref, w_ref, b_ref, o_ref):
        o_ref[0] = (
            jnp.dot(v_ref[...], w_ref[0], preferred_element_type=jnp.float32)
            + b_ref[0]
        )

    BR = 2528
    return pl.pallas_call(
        body,
        grid=(3, VPAD // BR),
        in_specs=[
            pl.BlockSpec((BR, D), lambda j, i: (i, 0)),
            pl.BlockSpec((1, D, D), lambda j, i: (j, 0, 0)),
            pl.BlockSpec((1, 1, D), lambda j, i: (j, 0, 0)),
        ],
        out_specs=pl.BlockSpec((1, BR, D), lambda j, i: (j, i, 0)),
        out_shape=jax.ShapeDtypeStruct((3, VPAD, D), jnp.float32),
    )(verts_pad, Wst, bst)


def _sc_body(of_hbm, ia_hbm, ib_hbm, out_hbm,
             acc, ia_v, ib_v, rows0, rows1, sem0, sem1, sem2, sem3):
    c = lax.axis_index("c")
    s = lax.axis_index("s")
    pltpu.sync_copy(of_hbm.at[pl.ds(c * VPAD + s * RPS, RPS)],
                    acc.at[pl.ds(s * RPS, RPS)])
    plsc.subcore_barrier()

    rows = (rows0, rows1)
    gsems = (sem0, sem1)
    ssems = (sem2, sem3)

    def blk(kb, carry):
        pltpu.sync_copy(ia_hbm.at[c, s, pl.ds(kb * KB, KB)], ia_v)
        pltpu.sync_copy(ib_hbm.at[c, s, pl.ds(kb * KB, KB)], ib_v)
        dg = {0: pltpu.async_copy(of_hbm.at[ib_v.at[0]], rows0, sem0)}
        ds = {}
        for j in range(KB):
            if j + 1 < KB:
                if j >= 1:
                    ds[j - 1].wait()
                dg[j + 1] = pltpu.async_copy(
                    of_hbm.at[ib_v.at[j + 1]], rows[(j + 1) % 2],
                    gsems[(j + 1) % 2])
            dg[j].wait()
            ds[j] = pltpu.async_copy(rows[j % 2], acc.at[ia_v.at[j]],
                                     ssems[j % 2], add=True)
        ds[KB - 2].wait()
        ds[KB - 1].wait()
        return carry

    lax.fori_loop(0, NBLK, blk, 0)
    plsc.subcore_barrier()
    pltpu.sync_copy(acc.at[pl.ds(s * RPS, RPS)],
                    out_hbm.at[c, pl.ds(s * RPS, RPS)])


def _sc_scatter(of, ia3, ib3):
    mesh = plsc.VectorSubcoreMesh(core_axis_name="c", subcore_axis_name="s")
    f = pl.kernel(
        _sc_body,
        out_type=jax.ShapeDtypeStruct((NCORE, VPAD, D), jnp.float32),
        mesh=mesh,
        scratch_types=[
            pltpu.VMEM_SHARED((VPAD, D), jnp.float32),
            pltpu.VMEM((KB, CHUNK), jnp.int32),
            pltpu.VMEM((KB, CHUNK), jnp.int32),
            pltpu.VMEM((CHUNK, D), jnp.float32),
            pltpu.VMEM((CHUNK, D), jnp.float32),
            pltpu.SemaphoreType.DMA,
            pltpu.SemaphoreType.DMA,
            pltpu.SemaphoreType.DMA,
            pltpu.SemaphoreType.DMA,
        ],
    )
    return f(of, ia3, ib3)


def _add(a, b):
    def body(a_ref, b_ref, o_ref):
        o_ref[...] = a_ref[...] + b_ref[...]

    BR = 2000
    return pl.pallas_call(
        body,
        grid=(V // BR,),
        in_specs=[
            pl.BlockSpec((BR, D), lambda i: (i, 0)),
            pl.BlockSpec((BR, D), lambda i: (i, 0)),
        ],
        out_specs=pl.BlockSpec((BR, D), lambda i: (i, 0)),
        out_shape=jax.ShapeDtypeStruct((V, D), jnp.float32),
    )(a, b)


def kernel(verts, edges, W0, b0, W1, b1):
    verts_pad = jnp.zeros((VPAD, D), jnp.float32).at[:V].set(verts)
    Wst = jnp.stack([W0, jnp.zeros_like(W0), W1])
    bst = jnp.stack([b0, jnp.zeros_like(b0), b1]).reshape(3, 1, D)
    Of = _matmuls(verts_pad, Wst, bst).reshape(3 * VPAD, D)

    e = edges.astype(jnp.int32)
    pad = jnp.full((EPAD - E,), V, jnp.int32)
    ia3 = jnp.stack([
        jnp.concatenate([e[:, 0], pad]),
        jnp.concatenate([e[:, 1], pad]),
    ]).reshape(NCORE, NSUB, KB * NBLK, CHUNK)
    ib3 = jnp.stack([
        jnp.concatenate([e[:, 1], pad]),
        jnp.concatenate([e[:, 0], pad]),
    ]).reshape(NCORE, NSUB, KB * NBLK, CHUNK) + 2 * VPAD

    partials = _sc_scatter(Of, ia3, ib3)
    return _add(partials[0, :V], partials[1, :V])
